# Initial kernel scaffold; baseline (speedup 1.0000x reference)
import jax
import jax.numpy as jnp
from jax.experimental import pallas as pl


def kernel(x, h0, wr, br, wu, bu, wo, bo):
    raise NotImplementedError("write your pallas kernel here")


# speedup vs baseline: 2.5829x; 2.5829x over previous
import functools

import jax
import jax.numpy as jnp
from jax.experimental import pallas as pl
from jax.experimental.pallas import tpu as pltpu


def _banded(w_hwio, W):
    _, _, cin, cout = w_hwio.shape
    bands = []
    for dy in range(3):
        acc = jnp.zeros((W, cin, W, cout), w_hwio.dtype)
        for dx in range(3):
            shift = jnp.eye(W, k=1 - dx, dtype=w_hwio.dtype)
            acc = acc + shift[:, None, :, None] * w_hwio[dy, dx][None, :, None, :]
        bands.append(acc.reshape(W * cin, W * cout))
    return jnp.stack(bands)


def _gru_kernel(xpad_ref, h0_ref, wx_ref, wru_ref, wo_ref, bias_ref, out_ref,
                hpad_ref, gpad_ref, h_ref, lx_ref, lh_ref, lg_ref, xp_ref,
                *, H, WC):
    t = pl.program_id(1)
    BB = out_ref.shape[0]
    M = BB * H

    @pl.when(t == 0)
    def _init():
        hpad_ref[...] = jnp.zeros(hpad_ref.shape, jnp.float32)
        gpad_ref[...] = jnp.zeros(gpad_ref.shape, jnp.float32)
        h0 = h0_ref[...]
        hpad_ref[:, 1:H + 1, :] = h0
        h_ref[...] = h0.reshape(M, WC)

    for dy in range(3):
        lx_ref[:, dy * WC:(dy + 1) * WC] = (
            xpad_ref[:, dy:dy + H, :].reshape(M, WC))
    xp_ref[...] = jnp.dot(lx_ref[...], wx_ref[...],
                          preferred_element_type=jnp.float32) + bias_ref[...]

    h_prev = h_ref[...]
    lh_ref[:, 0:WC] = hpad_ref[:, 0:H, :].reshape(M, WC)
    lh_ref[:, WC:2 * WC] = h_prev
    lh_ref[:, 2 * WC:] = hpad_ref[:, 2:H + 2, :].reshape(M, WC)
    acc_ru = jnp.dot(lh_ref[...], wru_ref[...],
                     preferred_element_type=jnp.float32) + xp_ref[:, :2 * WC]
    read_gate = jax.nn.sigmoid(acc_ru[:, :WC])
    update_gate = jax.nn.sigmoid(acc_ru[:, WC:])

    gated = read_gate * h_prev
    gpad_ref[:, 1:H + 1, :] = gated.reshape(BB, H, WC)
    lg_ref[:, 0:WC] = gpad_ref[:, 0:H, :].reshape(M, WC)
    lg_ref[:, WC:2 * WC] = gated
    lg_ref[:, 2 * WC:] = gpad_ref[:, 2:H + 2, :].reshape(M, WC)
    c = jnp.maximum(jnp.dot(lg_ref[...], wo_ref[...],
                            preferred_element_type=jnp.float32)
                    + xp_ref[:, 2 * WC:], 0.0)

    new_h = update_gate * h_prev + (1.0 - update_gate) * c
    h_ref[...] = new_h
    hpad_ref[:, 1:H + 1, :] = new_h.reshape(BB, H, WC)
    out_ref[...] = new_h.reshape(BB, H, WC)


def kernel(x, h0, wr, br, wu, bu, wo, bo):
    T, B, H, W, Cx = x.shape
    Ch = h0.shape[-1]
    WCx, WC = W * Cx, W * Ch
    NC = 2
    BB = B // NC
    M = BB * H

    wx = jnp.concatenate(
        [_banded(w[:, :, :Cx, :], W) for w in (wr, wu, wo)], axis=-1)
    wx = wx.reshape(3 * WCx, 3 * WC)
    wru = jnp.concatenate(
        [_banded(w[:, :, Cx:, :], W) for w in (wr, wu)], axis=-1)
    wru = wru.reshape(3 * WC, 2 * WC)
    wo_h = _banded(wo[:, :, Cx:, :], W).reshape(3 * WC, WC)
    bias = jnp.concatenate(
        [jnp.tile(b, W) for b in (br, bu, bo)]).reshape(1, 3 * WC)

    xpad = jnp.pad(x.reshape(T, B, H, WCx), ((0, 0), (0, 0), (1, 1), (0, 0)))
    xpad = xpad.reshape(T, NC, BB, H + 2, WCx)
    h0r = h0.reshape(NC, BB, H, WC)

    out = pl.pallas_call(
        functools.partial(_gru_kernel, H=H, WC=WC),
        out_shape=jax.ShapeDtypeStruct((T, NC, BB, H, WC), x.dtype),
        grid=(NC, T),
        in_specs=[
            pl.BlockSpec((None, None, BB, H + 2, WCx),
                         lambda c, t: (t, c, 0, 0, 0)),
            pl.BlockSpec((None, BB, H, WC), lambda c, t: (c, 0, 0, 0)),
            pl.BlockSpec((3 * WCx, 3 * WC), lambda c, t: (0, 0)),
            pl.BlockSpec((3 * WC, 2 * WC), lambda c, t: (0, 0)),
            pl.BlockSpec((3 * WC, WC), lambda c, t: (0, 0)),
            pl.BlockSpec((1, 3 * WC), lambda c, t: (0, 0)),
        ],
        out_specs=pl.BlockSpec((None, None, BB, H, WC),
                               lambda c, t: (t, c, 0, 0, 0)),
        scratch_shapes=[
            pltpu.VMEM((BB, H + 2, WC), jnp.float32),
            pltpu.VMEM((BB, H + 2, WC), jnp.float32),
            pltpu.VMEM((M, WC), jnp.float32),
            pltpu.VMEM((M, 3 * WCx), jnp.float32),
            pltpu.VMEM((M, 3 * WC), jnp.float32),
            pltpu.VMEM((M, 3 * WC), jnp.float32),
            pltpu.VMEM((M, 3 * WC), jnp.float32),
        ],
        compiler_params=pltpu.CompilerParams(
            dimension_semantics=("parallel", "arbitrary"),
            vmem_limit_bytes=100 * 1024 * 1024,
        ),
    )(xpad, h0r, wx, wru, wo_h, bias)

    return out.reshape(T, B, H, W, Ch)


# speedup vs baseline: 2.6958x; 1.0437x over previous
import functools

import jax
import jax.numpy as jnp
from jax.experimental import pallas as pl
from jax.experimental.pallas import tpu as pltpu


def _banded(w_hwio, W):
    _, _, cin, cout = w_hwio.shape
    bands = []
    for dy in range(3):
        acc = jnp.zeros((W, cin, W, cout), w_hwio.dtype)
        for dx in range(3):
            shift = jnp.eye(W, k=1 - dx, dtype=w_hwio.dtype)
            acc = acc + shift[:, None, :, None] * w_hwio[dy, dx][None, :, None, :]
        bands.append(acc.reshape(W * cin, W * cout))
    return jnp.stack(bands)


def _gru_kernel(x_ref, h0_ref, wx_ref, wru_ref, wo_ref, bias_ref, out_ref,
                h_ref, lx_ref, lh_ref, lg_ref, xp_ref, *, H, WC):
    t = pl.program_id(1)
    BB = out_ref.shape[0]
    M = BB * H
    WCx = x_ref.shape[-1]
    bf16 = jnp.bfloat16

    @pl.when(t == 0)
    def _init():
        lx_ref[...] = jnp.zeros(lx_ref.shape, bf16)
        lh_ref[...] = jnp.zeros(lh_ref.shape, bf16)
        lg_ref[...] = jnp.zeros(lg_ref.shape, bf16)
        h_ref[...] = h0_ref[...]

    xv = x_ref[...].astype(bf16)
    lx_ref[:, :, WCx:2 * WCx] = xv
    lx_ref[:, 1:H, 0:WCx] = xv[:, 0:H - 1, :]
    lx_ref[:, 0:H - 1, 2 * WCx:] = xv[:, 1:H, :]
    xp_ref[...] = jnp.dot(lx_ref[...].reshape(M, 3 * WCx), wx_ref[...],
                          preferred_element_type=jnp.float32) + bias_ref[...]

    hv = h_ref[...]
    hb = hv.astype(bf16)
    lh_ref[:, :, WC:2 * WC] = hb
    lh_ref[:, 1:H, 0:WC] = hb[:, 0:H - 1, :]
    lh_ref[:, 0:H - 1, 2 * WC:] = hb[:, 1:H, :]
    acc_ru = jnp.dot(lh_ref[...].reshape(M, 3 * WC), wru_ref[...],
                     preferred_element_type=jnp.float32) + xp_ref[:, :2 * WC]
    read_gate = jax.nn.sigmoid(acc_ru[:, :WC]).reshape(BB, H, WC)
    update_gate = jax.nn.sigmoid(acc_ru[:, WC:]).reshape(BB, H, WC)

    gated = (read_gate * hv).astype(bf16)
    lg_ref[:, :, WC:2 * WC] = gated
    lg_ref[:, 1:H, 0:WC] = gated[:, 0:H - 1, :]
    lg_ref[:, 0:H - 1, 2 * WC:] = gated[:, 1:H, :]
    c = jnp.maximum(jnp.dot(lg_ref[...].reshape(M, 3 * WC), wo_ref[...],
                            preferred_element_type=jnp.float32)
                    + xp_ref[:, 2 * WC:], 0.0).reshape(BB, H, WC)

    new_h = update_gate * hv + (1.0 - update_gate) * c
    h_ref[...] = new_h
    out_ref[...] = new_h


def kernel(x, h0, wr, br, wu, bu, wo, bo):
    T, B, H, W, Cx = x.shape
    Ch = h0.shape[-1]
    WCx, WC = W * Cx, W * Ch
    NC = 2
    BB = B // NC
    M = BB * H
    bf16 = jnp.bfloat16

    wrb, wub, wob = wr.astype(bf16), wu.astype(bf16), wo.astype(bf16)
    wx = jnp.concatenate(
        [_banded(w[:, :, :Cx, :], W) for w in (wrb, wub, wob)], axis=-1)
    wx = wx.reshape(3 * WCx, 3 * WC)
    wru = jnp.concatenate(
        [_banded(w[:, :, Cx:, :], W) for w in (wrb, wub)], axis=-1)
    wru = wru.reshape(3 * WC, 2 * WC)
    wo_h = _banded(wob[:, :, Cx:, :], W).reshape(3 * WC, WC)
    bias = jnp.concatenate(
        [jnp.tile(b, W) for b in (br, bu, bo)]).reshape(1, 3 * WC)

    xr = x.reshape(T, NC, BB, H, WCx)
    h0r = h0.reshape(NC, BB, H, WC)

    out = pl.pallas_call(
        functools.partial(_gru_kernel, H=H, WC=WC),
        out_shape=jax.ShapeDtypeStruct((T, NC, BB, H, WC), x.dtype),
        grid=(NC, T),
        in_specs=[
            pl.BlockSpec((None, None, BB, H, WCx),
                         lambda c, t: (t, c, 0, 0, 0)),
            pl.BlockSpec((None, BB, H, WC), lambda c, t: (c, 0, 0, 0)),
            pl.BlockSpec((3 * WCx, 3 * WC), lambda c, t: (0, 0)),
            pl.BlockSpec((3 * WC, 2 * WC), lambda c, t: (0, 0)),
            pl.BlockSpec((3 * WC, WC), lambda c, t: (0, 0)),
            pl.BlockSpec((1, 3 * WC), lambda c, t: (0, 0)),
        ],
        out_specs=pl.BlockSpec((None, None, BB, H, WC),
                               lambda c, t: (t, c, 0, 0, 0)),
        scratch_shapes=[
            pltpu.VMEM((BB, H, WC), jnp.float32),
            pltpu.VMEM((BB, H, 3 * WCx), bf16),
            pltpu.VMEM((BB, H, 3 * WC), bf16),
            pltpu.VMEM((BB, H, 3 * WC), bf16),
            pltpu.VMEM((M, 3 * WC), jnp.float32),
        ],
        compiler_params=pltpu.CompilerParams(
            dimension_semantics=("parallel", "arbitrary"),
            vmem_limit_bytes=100 * 1024 * 1024,
        ),
    )(xr, h0r, wx, wru, wo_h, bias)

    return out.reshape(T, B, H, W, Ch)


# speedup vs baseline: 3.5281x; 1.3087x over previous
import functools

import jax
import jax.numpy as jnp
from jax import lax
from jax.experimental import pallas as pl
from jax.experimental.pallas import tpu as pltpu

_BF16 = jnp.bfloat16


def _tile_mask_build(w_ref, base, cin0, cin, ch, W, wc, dst, gates):
    wcin = W * cin
    pt = (lax.broadcasted_iota(jnp.int32, (wcin, cin), 0) % cin ==
          lax.broadcasted_iota(jnp.int32, (wcin, cin), 1)).astype(_BF16)
    p = (lax.broadcasted_iota(jnp.int32, (ch, wc), 1) % ch ==
         lax.broadcasted_iota(jnp.int32, (ch, wc), 0)).astype(_BF16)
    rb = lax.broadcasted_iota(jnp.int32, (wcin, wc), 0) // cin
    cb = lax.broadcasted_iota(jnp.int32, (wcin, wc), 1) // ch
    diag = rb - cb + 1
    for gi, g in enumerate(gates):
        for dy in range(3):
            acc = jnp.zeros((wcin, wc), _BF16)
            for dx in range(3):
                w16 = w_ref[(base + g) * 9 + dy * 3 + dx][
                    cin0:cin0 + cin, :].astype(_BF16)
                tiled = jnp.dot(
                    pt, jnp.dot(w16, p, preferred_element_type=jnp.float32
                                ).astype(_BF16),
                    preferred_element_type=jnp.float32).astype(_BF16)
                acc = acc + jnp.where(diag == dx, tiled, jnp.zeros_like(tiled))
            dst[dy * wcin:(dy + 1) * wcin, gi * wc:(gi + 1) * wc] = acc


def _gru_kernel(x_ref, h0_ref, w_ref, bias_ref, out_ref,
                h_ref, lx_ref, lh_ref, lg_ref, xp_ref,
                wx_ref, wru_ref, wo_ref, *, H, W, Cx, Ch, WC):
    t = pl.program_id(1)
    BB = out_ref.shape[0]
    M = BB * H
    WCx = x_ref.shape[-1]

    @pl.when(t == 0)
    def _init():
        _tile_mask_build(w_ref, 0, 0, Cx, Ch, W, WC, wx_ref, (0, 1, 2))
        _tile_mask_build(w_ref, 0, Cx, Ch, Ch, W, WC, wru_ref, (0, 1))
        _tile_mask_build(w_ref, 2, Cx, Ch, Ch, W, WC, wo_ref, (0,))
        lx_ref[...] = jnp.zeros(lx_ref.shape, _BF16)
        lh_ref[...] = jnp.zeros(lh_ref.shape, _BF16)
        lg_ref[...] = jnp.zeros(lg_ref.shape, _BF16)
        h_ref[...] = h0_ref[...]

    xv = x_ref[...].astype(_BF16)
    lx_ref[:, :, WCx:2 * WCx] = xv
    lx_ref[:, 1:H, 0:WCx] = xv[:, 0:H - 1, :]
    lx_ref[:, 0:H - 1, 2 * WCx:] = xv[:, 1:H, :]
    xp_ref[...] = jnp.dot(lx_ref[...].reshape(M, 3 * WCx), wx_ref[...],
                          preferred_element_type=jnp.float32) + bias_ref[...]

    hv = h_ref[...]
    hb = hv.astype(_BF16)
    lh_ref[:, :, WC:2 * WC] = hb
    lh_ref[:, 1:H, 0:WC] = hb[:, 0:H - 1, :]
    lh_ref[:, 0:H - 1, 2 * WC:] = hb[:, 1:H, :]
    acc_ru = jnp.dot(lh_ref[...].reshape(M, 3 * WC), wru_ref[...],
                     preferred_element_type=jnp.float32) + xp_ref[:, :2 * WC]
    read_gate = jax.nn.sigmoid(acc_ru[:, :WC]).reshape(BB, H, WC)
    update_gate = jax.nn.sigmoid(acc_ru[:, WC:]).reshape(BB, H, WC)

    gated = (read_gate * hv).astype(_BF16)
    lg_ref[:, :, WC:2 * WC] = gated
    lg_ref[:, 1:H, 0:WC] = gated[:, 0:H - 1, :]
    lg_ref[:, 0:H - 1, 2 * WC:] = gated[:, 1:H, :]
    c = jnp.maximum(jnp.dot(lg_ref[...].reshape(M, 3 * WC), wo_ref[...],
                            preferred_element_type=jnp.float32)
                    + xp_ref[:, 2 * WC:], 0.0).reshape(BB, H, WC)

    new_h = update_gate * hv + (1.0 - update_gate) * c
    h_ref[...] = new_h
    out_ref[...] = new_h


def kernel(x, h0, wr, br, wu, bu, wo, bo):
    T, B, H, W, Cx = x.shape
    Ch = h0.shape[-1]
    WCx, WC = W * Cx, W * Ch
    NC = 2
    BB = B // NC
    M = BB * H

    wcat = jnp.stack([wr, wu, wo]).reshape(27, Cx + Ch, Ch)
    bias = jnp.concatenate(
        [jnp.tile(b, W) for b in (br, bu, bo)]).reshape(1, 3 * WC)

    xr = x.reshape(T, NC, BB, H, WCx)
    h0r = h0.reshape(NC, BB, H, WC)

    out = pl.pallas_call(
        functools.partial(_gru_kernel, H=H, W=W, Cx=Cx, Ch=Ch, WC=WC),
        out_shape=jax.ShapeDtypeStruct((T, NC, BB, H, WC), x.dtype),
        grid=(NC, T),
        in_specs=[
            pl.BlockSpec((None, None, BB, H, WCx),
                         lambda c, t: (t, c, 0, 0, 0)),
            pl.BlockSpec((None, BB, H, WC), lambda c, t: (c, 0, 0, 0)),
            pl.BlockSpec((27, Cx + Ch, Ch), lambda c, t: (0, 0, 0)),
            pl.BlockSpec((1, 3 * WC), lambda c, t: (0, 0)),
        ],
        out_specs=pl.BlockSpec((None, None, BB, H, WC),
                               lambda c, t: (t, c, 0, 0, 0)),
        scratch_shapes=[
            pltpu.VMEM((BB, H, WC), jnp.float32),
            pltpu.VMEM((BB, H, 3 * WCx), _BF16),
            pltpu.VMEM((BB, H, 3 * WC), _BF16),
            pltpu.VMEM((BB, H, 3 * WC), _BF16),
            pltpu.VMEM((M, 3 * WC), jnp.float32),
            pltpu.VMEM((3 * WCx, 3 * WC), _BF16),
            pltpu.VMEM((3 * WC, 2 * WC), _BF16),
            pltpu.VMEM((3 * WC, WC), _BF16),
        ],
        compiler_params=pltpu.CompilerParams(
            dimension_semantics=("parallel", "arbitrary"),
            vmem_limit_bytes=100 * 1024 * 1024,
        ),
    )(xr, h0r, wcat, bias)

    return out.reshape(T, B, H, W, Ch)


# speedup vs baseline: 4.5016x; 1.2759x over previous
import functools

import jax
import jax.numpy as jnp
from jax import lax
from jax.experimental import pallas as pl
from jax.experimental.pallas import tpu as pltpu

_BF16 = jnp.bfloat16


def _tile_mask_build(w_ref, base, cin0, cin, ch, W, wc, dst, gates):
    wcin = W * cin
    pt = (lax.broadcasted_iota(jnp.int32, (wcin, cin), 0) // W ==
          lax.broadcasted_iota(jnp.int32, (wcin, cin), 1)).astype(_BF16)
    p = (lax.broadcasted_iota(jnp.int32, (ch, wc), 1) // W ==
         lax.broadcasted_iota(jnp.int32, (ch, wc), 0)).astype(_BF16)
    rb = lax.broadcasted_iota(jnp.int32, (wcin, wc), 0) % W
    cb = lax.broadcasted_iota(jnp.int32, (wcin, wc), 1) % W
    diag = rb - cb + 1
    for gi, g in enumerate(gates):
        for dy in range(3):
            acc = jnp.zeros((wcin, wc), _BF16)
            for dx in range(3):
                w16 = w_ref[(base + g) * 9 + dy * 3 + dx][
                    cin0:cin0 + cin, :].astype(_BF16)
                tiled = jnp.dot(
                    pt, jnp.dot(w16, p, preferred_element_type=jnp.float32
                                ).astype(_BF16),
                    preferred_element_type=jnp.float32).astype(_BF16)
                acc = acc + jnp.where(diag == dx, tiled, jnp.zeros_like(tiled))
            dst[dy * wcin:(dy + 1) * wcin, gi * wc:(gi + 1) * wc] = acc


def _gru_kernel(x_ref, h0_ref, w_ref, bias_ref, out_ref,
                h_ref, lx_ref, lh_ref, lg_ref, xp_ref,
                wx_ref, wru_ref, wo_ref, *, H, W, Cx, Ch, WC):
    t = pl.program_id(1)
    BB = out_ref.shape[0]
    M = BB * H
    WCx = x_ref.shape[-1]

    @pl.when(t == 0)
    def _init():
        _tile_mask_build(w_ref, 0, 0, Cx, Ch, W, WC, wx_ref, (0, 1, 2))
        _tile_mask_build(w_ref, 0, Cx, Ch, Ch, W, WC, wru_ref, (0, 1))
        _tile_mask_build(w_ref, 2, Cx, Ch, Ch, W, WC, wo_ref, (0,))
        lx_ref[...] = jnp.zeros(lx_ref.shape, _BF16)
        lh_ref[...] = jnp.zeros(lh_ref.shape, _BF16)
        lg_ref[...] = jnp.zeros(lg_ref.shape, _BF16)
        h_ref[...] = h0_ref[...]

    xv = x_ref[...].astype(_BF16)
    lx_ref[:, :, WCx:2 * WCx] = xv
    lx_ref[:, 1:H, 0:WCx] = xv[:, 0:H - 1, :]
    lx_ref[:, 0:H - 1, 2 * WCx:] = xv[:, 1:H, :]
    xp_ref[...] = jnp.dot(lx_ref[...].reshape(M, 3 * WCx), wx_ref[...],
                          preferred_element_type=jnp.float32) + bias_ref[...]

    hv = h_ref[...]
    hb = hv.astype(_BF16)
    lh_ref[:, :, WC:2 * WC] = hb
    lh_ref[:, 1:H, 0:WC] = hb[:, 0:H - 1, :]
    lh_ref[:, 0:H - 1, 2 * WC:] = hb[:, 1:H, :]
    acc_ru = jnp.dot(lh_ref[...].reshape(M, 3 * WC), wru_ref[...],
                     preferred_element_type=jnp.float32) + xp_ref[:, :2 * WC]
    read_gate = jax.nn.sigmoid(acc_ru[:, :WC]).reshape(BB, H, WC)
    update_gate = jax.nn.sigmoid(acc_ru[:, WC:]).reshape(BB, H, WC)

    gated = (read_gate * hv).astype(_BF16)
    lg_ref[:, :, WC:2 * WC] = gated
    lg_ref[:, 1:H, 0:WC] = gated[:, 0:H - 1, :]
    lg_ref[:, 0:H - 1, 2 * WC:] = gated[:, 1:H, :]
    c = jnp.maximum(jnp.dot(lg_ref[...].reshape(M, 3 * WC), wo_ref[...],
                            preferred_element_type=jnp.float32)
                    + xp_ref[:, 2 * WC:], 0.0).reshape(BB, H, WC)

    new_h = update_gate * hv + (1.0 - update_gate) * c
    h_ref[...] = new_h
    out_ref[...] = new_h


def kernel(x, h0, wr, br, wu, bu, wo, bo):
    T, B, H, W, Cx = x.shape
    Ch = h0.shape[-1]
    WCx, WC = W * Cx, W * Ch
    NC = 2
    BB = B // NC
    M = BB * H

    wcat = jnp.stack([wr, wu, wo]).reshape(27, Cx + Ch, Ch)
    bias = jnp.concatenate(
        [jnp.repeat(b, W) for b in (br, bu, bo)]).reshape(1, 3 * WC)

    xr = jnp.transpose(x, (0, 1, 2, 4, 3)).reshape(T, NC, BB, H, WCx)
    h0r = jnp.transpose(h0, (0, 1, 3, 2)).reshape(NC, BB, H, WC)

    out = pl.pallas_call(
        functools.partial(_gru_kernel, H=H, W=W, Cx=Cx, Ch=Ch, WC=WC),
        out_shape=jax.ShapeDtypeStruct((T, NC, BB, H, WC), x.dtype),
        grid=(NC, T),
        in_specs=[
            pl.BlockSpec((None, None, BB, H, WCx),
                         lambda c, t: (t, c, 0, 0, 0)),
            pl.BlockSpec((None, BB, H, WC), lambda c, t: (c, 0, 0, 0)),
            pl.BlockSpec((27, Cx + Ch, Ch), lambda c, t: (0, 0, 0)),
            pl.BlockSpec((1, 3 * WC), lambda c, t: (0, 0)),
        ],
        out_specs=pl.BlockSpec((None, None, BB, H, WC),
                               lambda c, t: (t, c, 0, 0, 0)),
        scratch_shapes=[
            pltpu.VMEM((BB, H, WC), jnp.float32),
            pltpu.VMEM((BB, H, 3 * WCx), _BF16),
            pltpu.VMEM((BB, H, 3 * WC), _BF16),
            pltpu.VMEM((BB, H, 3 * WC), _BF16),
            pltpu.VMEM((M, 3 * WC), jnp.float32),
            pltpu.VMEM((3 * WCx, 3 * WC), _BF16),
            pltpu.VMEM((3 * WC, 2 * WC), _BF16),
            pltpu.VMEM((3 * WC, WC), _BF16),
        ],
        compiler_params=pltpu.CompilerParams(
            dimension_semantics=("parallel", "arbitrary"),
            vmem_limit_bytes=100 * 1024 * 1024,
        ),
    )(xr, h0r, wcat, bias)

    return jnp.transpose(out.reshape(T, B, H, Ch, W), (0, 1, 2, 4, 3))
